# baseline (speedup 1.0000x reference)
import functools

import jax
import jax.numpy as jnp
from jax import lax
from jax.experimental import pallas as pl
from jax.experimental.pallas import tpu as pltpu
from jax.experimental.pallas import tpu_sc as plsc

VOCAB = 1000000
EMBED = 64
BATCH = 4096
SEQ = 200

QPR = 4
REAL = SEQ // QPR
GLEN = 56
NC, NS = 2, 16
NW = NC * NS
ROWS_PER_W = BATCH // NW
Q_PER_W = ROWS_PER_W * QPR
NBUF = 16


def _reduce_group(buf, accs):
    def body(j, a):
        new = list(a)
        for r in (2 * j, 2 * j + 1):
            for c in range(EMBED // 16):
                new[c] = new[c] + buf[r, pl.ds(c * 16, 16)]
        return tuple(new)

    return lax.fori_loop(0, REAL // 2, body, accs)


@functools.partial(
    pl.kernel,
    mesh=plsc.VectorSubcoreMesh(core_axis_name="c", subcore_axis_name="s"),
    out_type=jax.ShapeDtypeStruct((BATCH, EMBED), jnp.float32),
    scratch_types=[
        pltpu.VMEM((Q_PER_W, GLEN), jnp.int32),
        pltpu.VMEM((ROWS_PER_W, EMBED), jnp.float32),
    ] + [pltpu.VMEM((GLEN, EMBED), jnp.float32) for _ in range(NBUF)]
      + [pltpu.SemaphoreType.DMA for _ in range(NBUF)],
    compiler_params=pltpu.CompilerParams(use_tc_tiling_on_sc=False),
)
def _sc_pool(idx_hbm, table_hbm, out_hbm, idx_v, out_v, *bufs_and_sems):
    bufs = bufs_and_sems[:NBUF]
    sems = bufs_and_sems[NBUF:]
    wid = lax.axis_index("s") * NC + lax.axis_index("c")
    base_q = wid * Q_PER_W
    base_r = wid * ROWS_PER_W

    pltpu.sync_copy(idx_hbm.at[pl.ds(base_q, Q_PER_W)], idx_v)

    def start(b, ql):
        pltpu.async_copy(table_hbm.at[idx_v.at[ql]], bufs[b], sems[b])

    def wait(b):
        pltpu.make_async_copy(
            table_hbm.at[idx_v.at[0]], bufs[b], sems[b]).wait()

    for b in range(NBUF):
        start(b, b)

    zeros = (jnp.zeros((16,), jnp.float32),) * (EMBED // 16)

    def step(i, carry):
        accs = zeros
        for b in range(NBUF):
            ql = NBUF * i + b
            wait(b)
            accs = _reduce_group(bufs[b], accs)
            if b % QPR == QPR - 1:
                row = QPR * i + b // QPR
                for c in range(EMBED // 16):
                    out_v[row, pl.ds(c * 16, 16)] = accs[c]
                accs = zeros
            start(b, jnp.minimum(ql + NBUF, Q_PER_W - 1))
        return carry

    lax.fori_loop(0, Q_PER_W // NBUF, step, 0)

    for b in range(NBUF):
        wait(b)

    pltpu.sync_copy(out_v, out_hbm.at[pl.ds(base_r, ROWS_PER_W)])


def _mlp_body(ps_ref, w1_ref, b1_ref, w2_ref, b2_ref, o_ref):
    pooled = ps_ref[...] * (1.0 / SEQ)
    h = jnp.dot(pooled, w1_ref[...], preferred_element_type=jnp.float32)
    h = jnp.maximum(h + b1_ref[...], 0.0)
    z = jnp.sum(h * w2_ref[...], axis=1, keepdims=True) + b2_ref[...]
    o_ref[...] = 1.0 / (1.0 + jnp.exp(-z))


def _mlp(pooled_sum, W1, b1, W2, b2):
    return pl.pallas_call(
        _mlp_body,
        out_shape=jax.ShapeDtypeStruct((BATCH, 1), jnp.float32),
    )(pooled_sum, W1, b1.reshape(1, EMBED), W2.reshape(1, EMBED),
      b2.reshape(1, 1))


def kernel(x, table, W1, b1, W2, b2):
    xg = x.astype(jnp.int32).reshape(BATCH, QPR, REAL)
    xg = jnp.pad(xg, ((0, 0), (0, 0), (0, GLEN - REAL)))
    idx = xg.reshape(BATCH * QPR, GLEN)
    pooled_sum = _sc_pool(idx, table)
    return _mlp(pooled_sum, W1, b1, W2, b2)


# speedup vs baseline: 3.6590x; 3.6590x over previous
import functools

import jax
import jax.numpy as jnp
from jax import lax
from jax.experimental import pallas as pl
from jax.experimental.pallas import tpu as pltpu
from jax.experimental.pallas import tpu_sc as plsc

VOCAB = 1000000
EMBED = 64
BATCH = 4096
SEQ = 200

QPR = 4
REAL = SEQ // QPR
GLEN = 56
NC, NS = 2, 16
NW = NC * NS
ROWS_PER_W = BATCH // NW
Q_PER_W = ROWS_PER_W * QPR
NBUF = 16


def _reduce_group(buf, accs):
    def body(j, a):
        new = list(a)
        for r in (2 * j, 2 * j + 1):
            for c in range(EMBED // 16):
                new[c] = new[c] + buf[r, pl.ds(c * 16, 16)]
        return tuple(new)

    return lax.fori_loop(0, REAL // 2, body, accs)


@functools.partial(
    pl.kernel,
    mesh=plsc.VectorSubcoreMesh(core_axis_name="c", subcore_axis_name="s"),
    out_type=jax.ShapeDtypeStruct((BATCH, EMBED), jnp.float32),
    scratch_types=[
        pltpu.VMEM((Q_PER_W, GLEN), jnp.int32),
        pltpu.VMEM((ROWS_PER_W, EMBED), jnp.float32),
    ] + [pltpu.VMEM((GLEN, EMBED), jnp.float32) for _ in range(NBUF)]
      + [pltpu.SemaphoreType.DMA for _ in range(NBUF)],
    compiler_params=pltpu.CompilerParams(use_tc_tiling_on_sc=False),
)
def _sc_pool(idx_hbm, table_hbm, out_hbm, idx_v, out_v, *bufs_and_sems):
    bufs = bufs_and_sems[:NBUF]
    sems = bufs_and_sems[NBUF:]
    wid = lax.axis_index("s") * NC + lax.axis_index("c")
    base_q = wid * Q_PER_W
    base_r = wid * ROWS_PER_W

    pltpu.sync_copy(idx_hbm.at[pl.ds(base_q, Q_PER_W)], idx_v)

    def start(b, ql):
        pltpu.async_copy(table_hbm.at[idx_v.at[ql]], bufs[b], sems[b])

    def wait(b):
        pltpu.make_async_copy(
            table_hbm.at[idx_v.at[0]], bufs[b], sems[b]).wait()

    for b in range(NBUF):
        start(b, b)

    zeros = (jnp.zeros((16,), jnp.float32),) * (EMBED // 16)

    def step(i, carry):
        accs = zeros
        for b in range(NBUF):
            ql = NBUF * i + b
            wait(b)
            accs = _reduce_group(bufs[b], accs)
            if b % QPR == QPR - 1:
                row = QPR * i + b // QPR
                for c in range(EMBED // 16):
                    out_v[row, pl.ds(c * 16, 16)] = accs[c]
                accs = zeros
            start(b, jnp.minimum(ql + NBUF, Q_PER_W - 1))
        return carry

    lax.fori_loop(0, Q_PER_W // NBUF, step, 0)

    for b in range(NBUF):
        wait(b)

    pltpu.sync_copy(out_v, out_hbm.at[pl.ds(base_r, ROWS_PER_W)])


def _mlp_body(ps_ref, w1_ref, b1_ref, w2_ref, b2_ref, o_ref):
    pooled = ps_ref[...] * (1.0 / SEQ)
    h = jnp.dot(pooled, w1_ref[...], preferred_element_type=jnp.float32)
    h = jnp.maximum(h + b1_ref[...], 0.0)
    z = jnp.sum(h * w2_ref[...], axis=1, keepdims=True) + b2_ref[...]
    o_ref[...] = 1.0 / (1.0 + jnp.exp(-z))


def _mlp(pooled_sum, W1, b1, W2, b2):
    return pl.pallas_call(
        _mlp_body,
        out_shape=jax.ShapeDtypeStruct((BATCH, 1), jnp.float32),
    )(pooled_sum, W1, b1.reshape(1, EMBED), W2.reshape(1, EMBED),
      b2.reshape(1, 1))


def kernel(x, table, W1, b1, W2, b2):
    xg = x.astype(jnp.int32).reshape(BATCH, QPR, REAL)
    npad = GLEN - REAL
    pad = (jnp.arange(BATCH * QPR * npad, dtype=jnp.int32) % VOCAB)
    xg = jnp.concatenate([xg, pad.reshape(BATCH, QPR, npad)], axis=2)
    idx = xg.reshape(BATCH * QPR, GLEN)
    pooled_sum = _sc_pool(idx, table)
    return _mlp(pooled_sum, W1, b1, W2, b2)


# speedup vs baseline: 4.8419x; 1.3233x over previous
import functools

import jax
import jax.numpy as jnp
from jax import lax
from jax.experimental import pallas as pl
from jax.experimental.pallas import tpu as pltpu
from jax.experimental.pallas import tpu_sc as plsc

VOCAB = 1000000
EMBED = 64
BATCH = 4096
SEQ = 200

NC, NS = 2, 16
NW = NC * NS
ROWS_PER_W = BATCH // NW
RPB = 2
WROWS = RPB * SEQ
N_WIN = ROWS_PER_W // RPB
IDX_PER_W = ROWS_PER_W * SEQ
CHUNK = 4 * WROWS
NCHUNK = IDX_PER_W // CHUNK


@functools.partial(
    pl.kernel,
    mesh=plsc.VectorSubcoreMesh(core_axis_name="c", subcore_axis_name="s"),
    out_type=jax.ShapeDtypeStruct((BATCH, EMBED), jnp.float32),
    scratch_types=[
        pltpu.VMEM((ROWS_PER_W, EMBED), jnp.float32),
        pltpu.VMEM((CHUNK,), jnp.int32),
        pltpu.VMEM((WROWS, EMBED), jnp.float32),
        pltpu.VMEM((WROWS, EMBED), jnp.float32),
        pltpu.SemaphoreType.DMA,
        pltpu.SemaphoreType.DMA,
    ],
)
def _sc_pool(idx_hbm, table_hbm, out_hbm, out_v, idx_v, buf_a, buf_b,
             sem_a, sem_b):
    wid = lax.axis_index("s") * NC + lax.axis_index("c")
    base_i = wid * IDX_PER_W
    base_r = wid * ROWS_PER_W
    bufs = (buf_a, buf_b)
    sems = (sem_a, sem_b)

    def load_chunk(c):
        pltpu.sync_copy(idx_hbm.at[pl.ds(base_i + c * CHUNK, CHUNK)], idx_v)

    def issue_window(woff, bb):
        buf = bufs[bb]
        sem = sems[bb]

        def issue16(k, carry):
            v = idx_v[pl.ds(woff + k * 16, 16)]
            for l in range(16):
                r = v[l]
                pltpu.async_copy(
                    table_hbm.at[pl.ds(r, 1), :],
                    buf.at[pl.ds(k * 16 + l, 1), :],
                    sem)
            return carry

        lax.fori_loop(0, WROWS // 16, issue16, 0)

    def wait_window(bb):
        pltpu.make_async_copy(
            table_hbm.at[pl.ds(0, WROWS), :], bufs[bb], sems[bb]).wait()

    def reduce_window(bb, row0):
        buf = bufs[bb]
        for half in range(RPB):
            zero = jnp.zeros((16,), jnp.float32)

            def body(j, accs, _half=half):
                base = _half * SEQ + 2 * j
                new = list(accs)
                for r in (base, base + 1):
                    for c in range(EMBED // 16):
                        new[c] = new[c] + buf[r, pl.ds(c * 16, 16)]
                return tuple(new)

            accs = lax.fori_loop(0, SEQ // 2, body, (zero,) * (EMBED // 16))
            for c in range(EMBED // 16):
                out_v[row0 + half, pl.ds(c * 16, 16)] = accs[c]

    load_chunk(0)
    issue_window(0 * WROWS, 0)
    issue_window(1 * WROWS, 1)

    def step(i, carry):
        for b in range(2):
            win = 2 * i + b
            nxt = win + 2
            wait_window(b)
            reduce_window(b, RPB * win)

            @pl.when(jnp.logical_and(lax.rem(nxt, 4) == 0, nxt < N_WIN))
            def _():
                load_chunk(nxt // 4)

            @pl.when(nxt < N_WIN)
            def _():
                issue_window(lax.rem(nxt, 4) * WROWS, b)
        return carry

    lax.fori_loop(0, N_WIN // 2, step, 0)

    pltpu.sync_copy(out_v, out_hbm.at[pl.ds(base_r, ROWS_PER_W)])


def _mlp_body(ps_ref, w1_ref, b1_ref, w2_ref, b2_ref, o_ref):
    pooled = ps_ref[...] * (1.0 / SEQ)
    h = jnp.dot(pooled, w1_ref[...], preferred_element_type=jnp.float32)
    h = jnp.maximum(h + b1_ref[...], 0.0)
    z = jnp.sum(h * w2_ref[...], axis=1, keepdims=True) + b2_ref[...]
    o_ref[...] = 1.0 / (1.0 + jnp.exp(-z))


def _mlp(pooled_sum, W1, b1, W2, b2):
    return pl.pallas_call(
        _mlp_body,
        out_shape=jax.ShapeDtypeStruct((BATCH, 1), jnp.float32),
    )(pooled_sum, W1, b1.reshape(1, EMBED), W2.reshape(1, EMBED),
      b2.reshape(1, 1))


def kernel(x, table, W1, b1, W2, b2):
    idx = x.astype(jnp.int32).reshape(-1)
    pooled_sum = _sc_pool(idx, table)
    return _mlp(pooled_sum, W1, b1, W2, b2)


# speedup vs baseline: 5.0403x; 1.0410x over previous
import functools

import jax
import jax.numpy as jnp
from jax import lax
from jax.experimental import pallas as pl
from jax.experimental.pallas import tpu as pltpu
from jax.experimental.pallas import tpu_sc as plsc

VOCAB = 1000000
EMBED = 64
BATCH = 4096
SEQ = 200

NC, NS = 2, 16
NW = NC * NS
ROWS_PER_W = BATCH // NW
IDX_PER_W = ROWS_PER_W * SEQ
CHUNK = 8 * SEQ
NCHUNK = IDX_PER_W // CHUNK
NBUF = 3
NFULL = SEQ // 16
TAIL = SEQ - 16 * NFULL


@functools.partial(
    pl.kernel,
    mesh=plsc.VectorSubcoreMesh(core_axis_name="c", subcore_axis_name="s"),
    out_type=jax.ShapeDtypeStruct((BATCH, EMBED), jnp.float32),
    scratch_types=[
        pltpu.VMEM((ROWS_PER_W, EMBED), jnp.float32),
        pltpu.VMEM((CHUNK + 16,), jnp.int32),
    ] + [pltpu.VMEM((SEQ, EMBED), jnp.float32) for _ in range(NBUF)]
      + [pltpu.SemaphoreType.DMA for _ in range(NBUF)],
)
def _sc_pool(idx_hbm, table_hbm, out_hbm, out_v, idx_v, *bufs_and_sems):
    bufs = bufs_and_sems[:NBUF]
    sems = bufs_and_sems[NBUF:]
    wid = lax.axis_index("s") * NC + lax.axis_index("c")
    base_i = wid * IDX_PER_W
    base_r = wid * ROWS_PER_W

    def load_chunk(c):
        pltpu.sync_copy(
            idx_hbm.at[pl.ds(base_i + c * CHUNK, CHUNK + 16)], idx_v)

    def issue16(woff, k, buf, sem, n=16):
        v = idx_v[pl.ds(woff + k * 16, 16)]
        for l in range(n):
            r = v[l]
            pltpu.async_copy(
                table_hbm.at[pl.ds(r, 1), :],
                buf.at[pl.ds(k * 16 + l, 1), :],
                sem)

    def issue_window(woff, b):
        def body(k, carry):
            issue16(woff, k, bufs[b], sems[b])
            return carry
        lax.fori_loop(0, NFULL, body, 0)
        issue16(woff, NFULL, bufs[b], sems[b], n=TAIL)

    def wait_window(b):
        pltpu.make_async_copy(
            table_hbm.at[pl.ds(0, SEQ), :], bufs[b], sems[b]).wait()

    def consume_window(win, b_red, b_iss):
        nxt = win + 2
        woff = lax.rem(nxt, 8) * SEQ
        do_issue = nxt < ROWS_PER_W

        @pl.when(jnp.logical_and(lax.rem(nxt, 8) == 0, do_issue))
        def _():
            load_chunk(nxt // 8)

        wait_window(b_red)
        buf = bufs[b_red]
        zero = jnp.zeros((16,), jnp.float32)

        def body(k, accs):
            @pl.when(do_issue)
            def _():
                issue16(woff, k, bufs[b_iss], sems[b_iss])
            new = list(accs)
            for r in range(16):
                for c in range(EMBED // 16):
                    new[c] = new[c] + buf[k * 16 + r, pl.ds(c * 16, 16)]
            return tuple(new)

        accs = lax.fori_loop(0, NFULL, body, (zero,) * (EMBED // 16))

        @pl.when(do_issue)
        def _():
            issue16(woff, NFULL, bufs[b_iss], sems[b_iss], n=TAIL)
        accs = list(accs)
        for r in range(TAIL):
            for c in range(EMBED // 16):
                accs[c] = accs[c] + buf[NFULL * 16 + r, pl.ds(c * 16, 16)]
        for c in range(EMBED // 16):
            out_v[win, pl.ds(c * 16, 16)] = accs[c]

    load_chunk(0)
    issue_window(0 * SEQ, 0)
    issue_window(1 * SEQ, 1)

    def step(i, carry):
        for b in range(NBUF):
            win = NBUF * i + b
            consume_window(win, b, (b + 2) % NBUF)
        return carry

    lax.fori_loop(0, ROWS_PER_W // NBUF, step, 0)
    for w in range(NBUF * (ROWS_PER_W // NBUF), ROWS_PER_W):
        consume_window(w, w % NBUF, (w + 2) % NBUF)

    pltpu.sync_copy(out_v, out_hbm.at[pl.ds(base_r, ROWS_PER_W)])


def _mlp_body(ps_ref, w1_ref, b1_ref, w2_ref, b2_ref, o_ref):
    pooled = ps_ref[...] * (1.0 / SEQ)
    h = jnp.dot(pooled, w1_ref[...], preferred_element_type=jnp.float32)
    h = jnp.maximum(h + b1_ref[...], 0.0)
    z = jnp.sum(h * w2_ref[...], axis=1, keepdims=True) + b2_ref[...]
    o_ref[...] = 1.0 / (1.0 + jnp.exp(-z))


def _mlp(pooled_sum, W1, b1, W2, b2):
    return pl.pallas_call(
        _mlp_body,
        out_shape=jax.ShapeDtypeStruct((BATCH, 1), jnp.float32),
    )(pooled_sum, W1, b1.reshape(1, EMBED), W2.reshape(1, EMBED),
      b2.reshape(1, 1))


def kernel(x, table, W1, b1, W2, b2):
    idx = x.astype(jnp.int32).reshape(-1)
    idx = jnp.concatenate([idx, jnp.arange(16, dtype=jnp.int32)])
    pooled_sum = _sc_pool(idx, table)
    return _mlp(pooled_sum, W1, b1, W2, b2)


# speedup vs baseline: 5.0813x; 1.0081x over previous
import functools

import jax
import jax.numpy as jnp
from jax import lax
from jax.experimental import pallas as pl
from jax.experimental.pallas import tpu as pltpu
from jax.experimental.pallas import tpu_sc as plsc

VOCAB = 1000000
EMBED = 64
BATCH = 4096
SEQ = 200

NC, NS = 2, 16
NW = NC * NS
ROWS_PER_W = BATCH // NW
IDX_PER_W = ROWS_PER_W * SEQ
CHUNK = 8 * SEQ
NCHUNK = IDX_PER_W // CHUNK
NBUF = 4
NFULL = SEQ // 16
TAIL = SEQ - 16 * NFULL


@functools.partial(
    pl.kernel,
    mesh=plsc.VectorSubcoreMesh(core_axis_name="c", subcore_axis_name="s"),
    out_type=jax.ShapeDtypeStruct((BATCH, EMBED), jnp.float32),
    scratch_types=[
        pltpu.VMEM((ROWS_PER_W, EMBED), jnp.float32),
        pltpu.VMEM((CHUNK + 16,), jnp.int32),
    ] + [pltpu.VMEM((SEQ, EMBED), jnp.float32) for _ in range(NBUF)]
      + [pltpu.SemaphoreType.DMA for _ in range(NBUF)],
)
def _sc_pool(idx_hbm, table_hbm, out_hbm, out_v, idx_v, *bufs_and_sems):
    bufs = bufs_and_sems[:NBUF]
    sems = bufs_and_sems[NBUF:]
    wid = lax.axis_index("s") * NC + lax.axis_index("c")
    base_i = wid * IDX_PER_W
    base_r = wid * ROWS_PER_W

    def load_chunk(c):
        pltpu.sync_copy(
            idx_hbm.at[pl.ds(base_i + c * CHUNK, CHUNK + 16)], idx_v)

    def issue16(woff, k, buf, sem, n=16):
        v = idx_v[pl.ds(woff + k * 16, 16)]
        for l in range(n):
            r = v[l]
            pltpu.async_copy(
                table_hbm.at[pl.ds(r, 1), :],
                buf.at[pl.ds(k * 16 + l, 1), :],
                sem)

    def issue_window(woff, b):
        def body(k, carry):
            issue16(woff, k, bufs[b], sems[b])
            return carry
        lax.fori_loop(0, NFULL, body, 0)
        issue16(woff, NFULL, bufs[b], sems[b], n=TAIL)

    def wait_window(b):
        pltpu.make_async_copy(
            table_hbm.at[pl.ds(0, SEQ), :], bufs[b], sems[b]).wait()

    def consume_window(win, b_red, b_iss):
        nxt = win + 3
        woff = lax.rem(nxt, 8) * SEQ
        do_issue = nxt < ROWS_PER_W

        @pl.when(jnp.logical_and(lax.rem(nxt, 8) == 0, do_issue))
        def _():
            load_chunk(nxt // 8)

        wait_window(b_red)
        buf = bufs[b_red]
        zero = jnp.zeros((16,), jnp.float32)

        def body(k, accs):
            @pl.when(do_issue)
            def _():
                issue16(woff, k, bufs[b_iss], sems[b_iss])
            new = list(accs)
            for r in range(16):
                for c in range(EMBED // 16):
                    new[c] = new[c] + buf[k * 16 + r, pl.ds(c * 16, 16)]
            return tuple(new)

        accs = lax.fori_loop(0, NFULL, body, (zero,) * (EMBED // 16))

        @pl.when(do_issue)
        def _():
            issue16(woff, NFULL, bufs[b_iss], sems[b_iss], n=TAIL)
        accs = list(accs)
        for r in range(TAIL):
            for c in range(EMBED // 16):
                accs[c] = accs[c] + buf[NFULL * 16 + r, pl.ds(c * 16, 16)]
        for c in range(EMBED // 16):
            out_v[win, pl.ds(c * 16, 16)] = accs[c]

    load_chunk(0)
    issue_window(0 * SEQ, 0)
    issue_window(1 * SEQ, 1)
    issue_window(2 * SEQ, 2)

    def step(i, carry):
        for b in range(NBUF):
            win = NBUF * i + b
            consume_window(win, b, (b + 3) % NBUF)
        return carry

    lax.fori_loop(0, ROWS_PER_W // NBUF, step, 0)

    pltpu.sync_copy(out_v, out_hbm.at[pl.ds(base_r, ROWS_PER_W)])


def _mlp_body(ps_ref, w1_ref, b1_ref, w2_ref, b2_ref, o_ref):
    pooled = ps_ref[...] * (1.0 / SEQ)
    h = jnp.dot(pooled, w1_ref[...], preferred_element_type=jnp.float32)
    h = jnp.maximum(h + b1_ref[...], 0.0)
    z = jnp.sum(h * w2_ref[...], axis=1, keepdims=True) + b2_ref[...]
    o_ref[...] = 1.0 / (1.0 + jnp.exp(-z))


def _mlp(pooled_sum, W1, b1, W2, b2):
    return pl.pallas_call(
        _mlp_body,
        out_shape=jax.ShapeDtypeStruct((BATCH, 1), jnp.float32),
    )(pooled_sum, W1, b1.reshape(1, EMBED), W2.reshape(1, EMBED),
      b2.reshape(1, 1))


def kernel(x, table, W1, b1, W2, b2):
    idx = x.astype(jnp.int32).reshape(-1)
    idx = jnp.concatenate([idx, jnp.arange(16, dtype=jnp.int32)])
    pooled_sum = _sc_pool(idx, table)
    return _mlp(pooled_sum, W1, b1, W2, b2)
